# baseline (speedup 1.0000x reference)
import functools

import jax
import jax.numpy as jnp
from jax import lax
from jax.experimental import pallas as pl
from jax.experimental.pallas import tpu as pltpu
from jax.experimental.pallas import tpu_sc as plsc

_B = 16384
_D = 64
_NC = 2
_NS = 16
_NW = _NC * _NS
_BPW = _B // _NW
_CHUNK = 128
_NCH = _BPW // _CHUNK
_GRP = 16


def _tt_body(uids, iids, utab, itab, out, uidx, iidx, urows, irows, ovec,
             *sems):
    wid = lax.axis_index("s") * _NC + lax.axis_index("c")

    pltpu.sync_copy(uids.at[wid], uidx)
    pltpu.sync_copy(iids.at[wid], iidx)

    copies = []
    for j in range(_NCH):
        cu = pltpu.async_copy(utab.at[uidx.at[j]],
                              urows.at[pl.ds(j * _CHUNK, _CHUNK)], sems[j])
        ci = pltpu.async_copy(itab.at[iidx.at[j]],
                              irows.at[pl.ds(j * _CHUNK, _CHUNK)], sems[j])
        copies.append((cu, ci))

    lane = lax.iota(jnp.int32, 16)

    for j in range(_NCH):
        cu, ci = copies[j]
        cu.wait()
        ci.wait()

        def grp_body(g, carry, j=j):
            base = j * _CHUNK + g * _GRP
            row = base + lane
            accs = [jnp.zeros((_GRP,), jnp.float32) for _ in range(4)]
            for d in range(_D):
                col = jnp.full((_GRP,), d, jnp.int32)
                uv = plsc.load_gather(urows, [row, col])
                iv = plsc.load_gather(irows, [row, col])
                accs[d % 4] = accs[d % 4] + uv * iv
            acc = (accs[0] + accs[1]) + (accs[2] + accs[3])
            ovec[pl.ds(base, _GRP)] = acc
            return carry

        lax.fori_loop(0, _CHUNK // _GRP, grp_body, 0)

    pltpu.sync_copy(ovec, out.at[pl.ds(wid * _BPW, _BPW)])


@jax.jit
def _two_tower(uids, iids, utab, itab):
    mesh = plsc.VectorSubcoreMesh(core_axis_name="c", subcore_axis_name="s")
    run = functools.partial(
        pl.kernel,
        mesh=mesh,
        out_type=jax.ShapeDtypeStruct((_B,), jnp.float32),
        scratch_types=[
            pltpu.VMEM((_NCH, _CHUNK), jnp.int32),
            pltpu.VMEM((_NCH, _CHUNK), jnp.int32),
            pltpu.VMEM((_BPW, _D), jnp.float32),
            pltpu.VMEM((_BPW, _D), jnp.float32),
            pltpu.VMEM((_BPW,), jnp.float32),
        ] + [pltpu.SemaphoreType.DMA] * _NCH,
        compiler_params=pltpu.CompilerParams(
            needs_layout_passes=False, use_tc_tiling_on_sc=False),
    )(_tt_body)
    return run(uids, iids, utab, itab)


def kernel(user_ids, item_ids, user_table, item_table):
    uids = user_ids.astype(jnp.int32).reshape(_NW, _NCH, _CHUNK)
    iids = item_ids.astype(jnp.int32).reshape(_NW, _NCH, _CHUNK)
    return _two_tower(uids, iids, user_table, item_table)
